# Initial kernel scaffold; baseline (speedup 1.0000x reference)
import jax
import jax.numpy as jnp
from jax.experimental import pallas as pl


def kernel(q, k, v, gate, eta, W0, gamma, beta):
    raise NotImplementedError("write your pallas kernel here")


# speedup vs baseline: 107.9273x; 107.9273x over previous
import functools

import jax
import jax.numpy as jnp
from jax.experimental import pallas as pl
from jax.experimental.pallas import tpu as pltpu

EPS = 1e-6
_LOG_TINY = -88.0


def _titans_kernel(gcol_ref, grow_ref, eta_ref, q_ref, k_ref, v_ref,
                   w0_ref, gam_ref, bet_ref, o_ref, s_ref, *, nc):
    c = pl.program_id(1)

    @pl.when(c == 0)
    def _():
        s_ref[...] = w0_ref[0]

    qc = q_ref[0]
    kc = k_ref[0]
    vc = v_ref[0]
    w0 = w0_ref[0]
    gam = gam_ref[0]
    bet = bet_ref[0]
    eta = eta_ref[0, 0]

    cdim = qc.shape[0]
    f32 = jnp.float32

    z1 = jnp.dot(kc, w0, preferred_element_type=f32)
    mu = jnp.mean(z1, axis=-1, keepdims=True)
    var = jnp.mean((z1 - mu) ** 2, axis=-1, keepdims=True)
    std = jnp.sqrt(var + EPS)
    x_hat = (z1 - mu) / std
    y = gam * x_hat + bet + kc
    gxh = (2.0 * (y - vc)) * gam
    z = (gxh - jnp.mean(gxh, axis=-1, keepdims=True)
         - x_hat * jnp.mean(gxh * x_hat, axis=-1, keepdims=True)) / std
    u = (-eta) * z

    lg_col = jnp.maximum(jnp.log(gcol_ref[0, 0]), _LOG_TINY)
    lg_row = jnp.maximum(jnp.log(grow_ref[0, 0]), _LOG_TINY)
    ii = jax.lax.broadcasted_iota(jnp.int32, (cdim, cdim), 0)
    jj = jax.lax.broadcasted_iota(jnp.int32, (cdim, cdim), 1)
    tri_low = (ii >= jj).astype(f32)
    cl_col = jnp.dot(tri_low, lg_col, preferred_element_type=f32)
    cl_row = jnp.dot(lg_row, (ii <= jj).astype(f32),
                     preferred_element_type=f32)
    sum_lg = jnp.sum(lg_col)

    diff = jnp.where(ii >= jj, cl_col - cl_row, _LOG_TINY * 100.0)
    amat = jnp.exp(diff)

    s_old = s_ref[...]
    inter = jnp.exp(cl_col) * jnp.dot(qc, s_old, preferred_element_type=f32)
    qk = jax.lax.dot_general(qc, kc, (((1,), (1,)), ((), ())),
                             preferred_element_type=f32)
    zq = inter + jnp.dot(qk * amat, u, preferred_element_type=f32)

    @pl.when(c < nc - 1)
    def _():
        b_col = jnp.exp(sum_lg - cl_col)
        s_ref[...] = (jnp.exp(sum_lg) * s_old
                      + jax.lax.dot_general(b_col * kc, u,
                                            (((0,), (0,)), ((), ())),
                                            preferred_element_type=f32))

    mu2 = jnp.mean(zq, axis=-1, keepdims=True)
    var2 = jnp.mean((zq - mu2) ** 2, axis=-1, keepdims=True)
    zq_hat = (zq - mu2) / jnp.sqrt(var2 + EPS)
    o_ref[0] = gam * zq_hat + bet + qc


@functools.partial(jax.jit, static_argnames=("chunk",))
def _run(q, k, v, gate, eta, w0, gamma, beta, chunk=128):
    b, h, l, d = q.shape
    bh = b * h
    nc = l // chunk

    qf = q.reshape(bh, l, d)
    kf = k.reshape(bh, l, d)
    vf = v.reshape(bh, l, d)
    g_col = gate.reshape(bh, nc, chunk, 1)
    g_row = gate.reshape(bh, nc, 1, chunk)
    eta_col = eta.reshape(bh, nc, chunk, 1)
    w0f = jnp.broadcast_to(w0[None], (b, h, d, d)).reshape(bh, d, d)
    gamf = jnp.broadcast_to(gamma[None], (b, h, 1, d)).reshape(bh, 1, d)
    betf = jnp.broadcast_to(beta[None], (b, h, 1, d)).reshape(bh, 1, d)

    seq_spec = pl.BlockSpec((1, chunk, d), lambda i, c: (i, c, 0))
    col_spec = pl.BlockSpec((1, 1, chunk, 1), lambda i, c: (i, c, 0, 0))
    row_spec = pl.BlockSpec((1, 1, 1, chunk), lambda i, c: (i, c, 0, 0))
    head_mat = pl.BlockSpec((1, d, d), lambda i, c: (i, 0, 0))
    head_vec = pl.BlockSpec((1, 1, d), lambda i, c: (i, 0, 0))

    out = pl.pallas_call(
        functools.partial(_titans_kernel, nc=nc),
        out_shape=jax.ShapeDtypeStruct((bh, l, d), jnp.float32),
        grid=(bh, nc),
        in_specs=[col_spec, row_spec, col_spec, seq_spec, seq_spec, seq_spec,
                  head_mat, head_vec, head_vec],
        out_specs=seq_spec,
        scratch_shapes=[pltpu.VMEM((d, d), jnp.float32)],
        compiler_params=pltpu.CompilerParams(
            dimension_semantics=("parallel", "arbitrary"),
        ),
        name="titans_memory_gla",
    )(g_col, g_row, eta_col, qf, kf, vf, w0f, gamf, betf)
    return out.reshape(b, h, l, d)


def kernel(q, k, v, gate, eta, W0, gamma, beta):
    return _run(q, k, v, gate, eta, W0, gamma, beta)


# speedup vs baseline: 119.0830x; 1.1034x over previous
import functools

import jax
import jax.numpy as jnp
from jax.experimental import pallas as pl
from jax.experimental.pallas import tpu as pltpu

EPS = 1e-6
_LOG_TINY = -88.0


def _titans_kernel(g_ref, e_ref, q_ref, k_ref, v_ref,
                   w0_ref, gam_ref, bet_ref, o_ref, s_ref, *, nc):
    c = pl.program_id(1)

    @pl.when(c == 0)
    def _():
        s_ref[...] = w0_ref[0]

    qc = q_ref[0]
    kc = k_ref[0]
    vc = v_ref[0]
    w0 = w0_ref[0]
    gam = gam_ref[0]
    bet = bet_ref[0]
    eta_row = e_ref[0]

    cdim = qc.shape[0]
    f32 = jnp.float32
    dn_t = (((1,), (1,)), ((), ()))
    dn_r = (((0,), (0,)), ((), ()))

    z1 = jnp.dot(kc, w0, preferred_element_type=f32)
    mu = jnp.mean(z1, axis=-1, keepdims=True)
    var = jnp.mean((z1 - mu) ** 2, axis=-1, keepdims=True)
    std = jnp.sqrt(var + EPS)
    x_hat = (z1 - mu) / std
    y = gam * x_hat + bet + kc
    gxh = (2.0 * (y - vc)) * gam
    z = (gxh - jnp.mean(gxh, axis=-1, keepdims=True)
         - x_hat * jnp.mean(gxh * x_hat, axis=-1, keepdims=True)) / std

    lg_row = jnp.maximum(jnp.log(g_ref[0]), _LOG_TINY)
    ii = jax.lax.broadcasted_iota(jnp.int32, (cdim, cdim), 0)
    jj = jax.lax.broadcasted_iota(jnp.int32, (cdim, cdim), 1)
    tri_low = (ii >= jj).astype(f32)
    eye = (ii == jj).astype(f32)
    cl_col = jax.lax.dot_general(tri_low, lg_row, dn_t,
                                 preferred_element_type=f32)
    cl_row = jax.lax.dot_general(lg_row, tri_low, dn_t,
                                 preferred_element_type=f32)
    eta_col = jax.lax.dot_general(eye, eta_row, dn_t,
                                  preferred_element_type=f32)
    sum_lg = jnp.sum(lg_row)

    diff = jnp.where(ii >= jj, cl_col - cl_row, _LOG_TINY * 100.0)
    amat_eta = jnp.exp(diff) * (-eta_row)

    s_old = s_ref[...]
    inter = jnp.exp(cl_col) * jnp.dot(qc, s_old, preferred_element_type=f32)
    qk = jax.lax.dot_general(qc, kc, dn_t,
                             preferred_element_type=f32)
    zq = inter + jnp.dot(qk * amat_eta, z, preferred_element_type=f32)

    @pl.when(c < nc - 1)
    def _():
        be_col = jnp.exp(sum_lg - cl_col) * (-eta_col)
        s_ref[...] = (jnp.exp(sum_lg) * s_old
                      + jax.lax.dot_general(be_col * kc, z, dn_r,
                                            preferred_element_type=f32))

    mu2 = jnp.mean(zq, axis=-1, keepdims=True)
    var2 = jnp.mean((zq - mu2) ** 2, axis=-1, keepdims=True)
    zq_hat = (zq - mu2) / jnp.sqrt(var2 + EPS)
    o_ref[0] = gam * zq_hat + bet + qc


@functools.partial(jax.jit, static_argnames=("chunk",))
def _run(q, k, v, gate, eta, w0, gamma, beta, chunk=128):
    b, h, l, d = q.shape
    bh = b * h
    nc = l // chunk

    qf = q.reshape(bh, l, d)
    kf = k.reshape(bh, l, d)
    vf = v.reshape(bh, l, d)
    g_row = gate.reshape(bh * nc, 1, chunk)
    e_row = eta.reshape(bh * nc, 1, chunk)
    w0f = jnp.broadcast_to(w0[None], (b, h, d, d)).reshape(bh, d, d)
    gamf = jnp.broadcast_to(gamma[None], (b, h, 1, d)).reshape(bh, 1, d)
    betf = jnp.broadcast_to(beta[None], (b, h, 1, d)).reshape(bh, 1, d)

    seq_spec = pl.BlockSpec((1, chunk, d), lambda i, c: (i, c, 0))
    row_spec = pl.BlockSpec((1, 1, chunk), lambda i, c: (i * nc + c, 0, 0))
    head_mat = pl.BlockSpec((1, d, d), lambda i, c: (i, 0, 0))
    head_vec = pl.BlockSpec((1, 1, d), lambda i, c: (i, 0, 0))

    out = pl.pallas_call(
        functools.partial(_titans_kernel, nc=nc),
        out_shape=jax.ShapeDtypeStruct((bh, l, d), jnp.float32),
        grid=(bh, nc),
        in_specs=[row_spec, row_spec, seq_spec, seq_spec, seq_spec,
                  head_mat, head_vec, head_vec],
        out_specs=seq_spec,
        scratch_shapes=[pltpu.VMEM((d, d), jnp.float32)],
        compiler_params=pltpu.CompilerParams(
            dimension_semantics=("parallel", "arbitrary"),
        ),
        name="titans_memory_gla",
    )(g_row, e_row, qf, kf, vf, w0f, gamf, betf)
    return out.reshape(b, h, l, d)


def kernel(q, k, v, gate, eta, W0, gamma, beta):
    return _run(q, k, v, gate, eta, W0, gamma, beta)


# speedup vs baseline: 126.7331x; 1.0642x over previous
import functools

import jax
import jax.numpy as jnp
from jax.experimental import pallas as pl
from jax.experimental.pallas import tpu as pltpu

EPS = 1e-6
_LOG_TINY = -88.0


def _titans_kernel(g_ref, e_ref, q_ref, k_ref, v_ref,
                   w0_ref, gam_ref, bet_ref, o_ref, s_ref, *, nc, hb):
    c = pl.program_id(1)

    @pl.when(c == 0)
    def _():
        s_ref[...] = w0_ref[0]

    f32 = jnp.float32
    dn_t = (((1,), (1,)), ((), ()))
    dn_r = (((0,), (0,)), ((), ()))

    cdim = q_ref.shape[2]
    ii = jax.lax.broadcasted_iota(jnp.int32, (cdim, cdim), 0)
    jj = jax.lax.broadcasted_iota(jnp.int32, (cdim, cdim), 1)
    tri_low = (ii >= jj).astype(f32)
    eye = (ii == jj).astype(f32)
    causal = ii >= jj

    lg = jnp.maximum(jnp.log(g_ref[0, 0]), _LOG_TINY)
    cl = jax.lax.dot_general(lg, tri_low, dn_t,
                             preferred_element_type=f32)
    cl_cols = jax.lax.dot_general(tri_low, lg, dn_t,
                                  preferred_element_type=f32)
    eta_all = e_ref[0, 0]
    eta_cols = jax.lax.dot_general(eye, eta_all, dn_t,
                                   preferred_element_type=f32)
    sum_lg = jnp.sum(lg, axis=-1, keepdims=True)

    for j in range(hb):
        qc = q_ref[0, j]
        kc = k_ref[0, j]
        vc = v_ref[0, j]
        gam = gam_ref[0, j]
        bet = bet_ref[0, j]

        z1 = jnp.dot(kc, w0_ref[0, j], preferred_element_type=f32)
        mu = jnp.mean(z1, axis=-1, keepdims=True)
        var = jnp.mean((z1 - mu) ** 2, axis=-1, keepdims=True)
        std = jnp.sqrt(var + EPS)
        x_hat = (z1 - mu) / std
        y = gam * x_hat + bet + kc
        gxh = (2.0 * (y - vc)) * gam
        z = (gxh - jnp.mean(gxh, axis=-1, keepdims=True)
             - x_hat * jnp.mean(gxh * x_hat, axis=-1, keepdims=True)) / std

        cl_row = cl[j:j + 1, :]
        cl_col = cl_cols[:, j:j + 1]
        eta_row = eta_all[j:j + 1, :]
        eta_col = eta_cols[:, j:j + 1]

        diff = jnp.where(causal, cl_col - cl_row, _LOG_TINY * 100.0)
        amat_eta = jnp.exp(diff) * (-eta_row)

        s_old = s_ref[j]
        inter = jnp.exp(cl_col) * jnp.dot(qc, s_old,
                                          preferred_element_type=f32)
        qk = jax.lax.dot_general(qc, kc, dn_t,
                                 preferred_element_type=f32)
        zq = inter + jnp.dot(qk * amat_eta, z, preferred_element_type=f32)

        @pl.when(c < nc - 1)
        def _():
            be_col = jnp.exp(sum_lg[j, 0] - cl_col) * (-eta_col)
            s_ref[j] = (jnp.exp(sum_lg[j, 0]) * s_old
                        + jax.lax.dot_general(be_col * kc, z, dn_r,
                                              preferred_element_type=f32))

        mu2 = jnp.mean(zq, axis=-1, keepdims=True)
        var2 = jnp.mean((zq - mu2) ** 2, axis=-1, keepdims=True)
        zq_hat = (zq - mu2) / jnp.sqrt(var2 + EPS)
        o_ref[0, j] = gam * zq_hat + bet + qc


@functools.partial(jax.jit, static_argnames=("chunk", "hb"))
def _run(q, k, v, gate, eta, w0, gamma, beta, chunk=128, hb=4):
    b, h, l, d = q.shape
    bh = b * h
    hg = bh // hb
    nc = l // chunk

    qf = q.reshape(hg, hb, l, d)
    kf = k.reshape(hg, hb, l, d)
    vf = v.reshape(hg, hb, l, d)
    g_row = gate.reshape(hg, hb, nc, chunk).transpose(0, 2, 1, 3)
    e_row = eta.reshape(hg, hb, nc, chunk).transpose(0, 2, 1, 3)
    w0f = jnp.broadcast_to(w0[None], (b, h, d, d)).reshape(hg, hb, d, d)
    gamf = jnp.broadcast_to(gamma[None], (b, h, 1, d)).reshape(hg, hb, 1, d)
    betf = jnp.broadcast_to(beta[None], (b, h, 1, d)).reshape(hg, hb, 1, d)

    seq_spec = pl.BlockSpec((1, hb, chunk, d), lambda i, c: (i, 0, c, 0))
    row_spec = pl.BlockSpec((1, 1, hb, chunk), lambda i, c: (i, c, 0, 0))
    head_mat = pl.BlockSpec((1, hb, d, d), lambda i, c: (i, 0, 0, 0))
    head_vec = pl.BlockSpec((1, hb, 1, d), lambda i, c: (i, 0, 0, 0))

    out = pl.pallas_call(
        functools.partial(_titans_kernel, nc=nc, hb=hb),
        out_shape=jax.ShapeDtypeStruct((hg, hb, l, d), jnp.float32),
        grid=(hg, nc),
        in_specs=[row_spec, row_spec, seq_spec, seq_spec, seq_spec,
                  head_mat, head_vec, head_vec],
        out_specs=seq_spec,
        scratch_shapes=[pltpu.VMEM((hb, d, d), jnp.float32)],
        compiler_params=pltpu.CompilerParams(
            dimension_semantics=("parallel", "arbitrary"),
        ),
        name="titans_memory_gla",
    )(g_row, e_row, qf, kf, vf, w0f, gamf, betf)
    return out.reshape(b, h, l, d)


def kernel(q, k, v, gate, eta, W0, gamma, beta):
    return _run(q, k, v, gate, eta, W0, gamma, beta)
